# baseline (speedup 1.0000x reference)
import functools

import jax
import jax.numpy as jnp
from jax import lax
from jax.experimental import pallas as pl
from jax.experimental.pallas import tpu as pltpu
from jax.experimental.pallas import tpu_sc as plsc

EMB = 64
B = 4096
L = 201
NBEH = 200
LPAD = 208
HALF = 104
NW = 32
SPW = B // NW


def _pool_body(x_hbm, tab_hbm, out_hbm, xw, buf0, buf1, outw, t0, sem0, sem1):
    wid = lax.axis_index("s") * 2 + lax.axis_index("c")
    base = wid * SPW

    pltpu.sync_copy(x_hbm.at[pl.ds(base, SPW)], xw)
    pltpu.sync_copy(tab_hbm.at[pl.ds(0, 1)], t0)

    bufs = (buf0, buf1)
    sems = (sem0, sem1)

    def issue(s, b):
        pltpu.async_copy(tab_hbm.at[xw.at[s, pl.ds(0, HALF)]],
                         bufs[b].at[pl.ds(0, HALF)], sems[b])
        pltpu.async_copy(tab_hbm.at[xw.at[s, pl.ds(HALF, HALF)]],
                         bufs[b].at[pl.ds(HALF, HALF)], sems[b])

    def wait(b):
        pltpu.make_async_copy(tab_hbm.at[pl.ds(0, LPAD)], bufs[b], sems[b]).wait()

    def accumulate(s, b):
        buf = bufs[b]
        cnt = jnp.zeros((16,), jnp.int32)
        for c in range(12):
            ids = xw[s, pl.ds(16 * c, 16)]
            cnt = cnt + jnp.where(ids > 0, 1, 0)
        ids12 = xw[s, pl.ds(192, 16)]
        lane = lax.iota(jnp.int32, 16)
        cnt = cnt + jnp.where((ids12 > 0) & (lane < 8), 1, 0)
        nnz = jnp.sum(cnt)

        def row_body(r, acc):
            a = list(acc)
            for rr in range(4):
                row = r * 4 + rr
                for c in range(4):
                    a[c] = a[c] + buf[row, pl.ds(16 * c, 16)]
            return tuple(a)

        z = jnp.zeros((16,), jnp.float32)
        acc = lax.fori_loop(0, NBEH // 4, row_body, (z, z, z, z))

        avg = nnz.astype(jnp.float32) * (1.0 / NBEH)
        zero_scale = (NBEH - nnz).astype(jnp.float32) * avg
        for c in range(4):
            pooled = acc[c] * avg - t0[0, pl.ds(16 * c, 16)] * zero_scale
            outw[s, pl.ds(16 * c, 16)] = pooled
            outw[s, pl.ds(EMB + 16 * c, 16)] = buf[NBEH, pl.ds(16 * c, 16)]

    issue(0, 0)
    issue(1, 1)

    def outer(i, carry):
        s0 = i * 2
        for b in range(2):
            wait(b)
            accumulate(s0 + b, b)
            issue(s0 + b + 2, b)
        return carry

    lax.fori_loop(0, (SPW - 2) // 2, outer, 0)
    for b in range(2):
        wait(b)
        accumulate(SPW - 2 + b, b)

    pltpu.sync_copy(outw, out_hbm.at[pl.ds(base, SPW)])


_pool = pl.kernel(
    _pool_body,
    out_type=jax.ShapeDtypeStruct((B, 2 * EMB), jnp.float32),
    mesh=plsc.VectorSubcoreMesh(core_axis_name="c", subcore_axis_name="s",
                                num_cores=2, num_subcores=16),
    compiler_params=pltpu.CompilerParams(use_tc_tiling_on_sc=False,
                                         needs_layout_passes=False),
    scratch_types=[
        pltpu.VMEM((SPW, LPAD), jnp.int32),
        pltpu.VMEM((LPAD, EMB), jnp.float32),
        pltpu.VMEM((LPAD, EMB), jnp.float32),
        pltpu.VMEM((SPW, 2 * EMB), jnp.float32),
        pltpu.VMEM((1, EMB), jnp.float32),
        pltpu.SemaphoreType.DMA,
        pltpu.SemaphoreType.DMA,
    ],
)


def _mlp_body(f_ref, w1_ref, b1_ref, w2_ref, b2_ref, w3_ref, b3_ref, o_ref):
    h = jnp.dot(f_ref[...], w1_ref[...], preferred_element_type=jnp.float32)
    h = jnp.maximum(h + b1_ref[...], 0.0)
    h = jnp.dot(h, w2_ref[...], preferred_element_type=jnp.float32)
    h = jnp.maximum(h + b2_ref[...], 0.0)
    o = jnp.dot(h, w3_ref[...], preferred_element_type=jnp.float32) + b3_ref[...]
    o_ref[...] = 1.0 / (1.0 + jnp.exp(-o))


BM = 512


@functools.partial(jax.jit, static_argnames=())
def _mlp(feats, W1, b1, W2, b2, W3, b3):
    return pl.pallas_call(
        _mlp_body,
        grid=(B // BM,),
        in_specs=[
            pl.BlockSpec((BM, 2 * EMB), lambda i: (i, 0)),
            pl.BlockSpec(W1.shape, lambda i: (0, 0)),
            pl.BlockSpec((1, b1.shape[0]), lambda i: (0, 0)),
            pl.BlockSpec(W2.shape, lambda i: (0, 0)),
            pl.BlockSpec((1, b2.shape[0]), lambda i: (0, 0)),
            pl.BlockSpec(W3.shape, lambda i: (0, 0)),
            pl.BlockSpec((1, 1), lambda i: (0, 0)),
        ],
        out_specs=pl.BlockSpec((BM, 1), lambda i: (i, 0)),
        out_shape=jax.ShapeDtypeStruct((B, 1), jnp.float32),
    )(feats, W1, b1.reshape(1, -1), W2, b2.reshape(1, -1), W3, b3.reshape(1, 1))


def kernel(x, emb_table, W1, b1, W2, b2, W3, b3):
    xp = jnp.concatenate([x, jnp.zeros((B, LPAD - L), jnp.int32)], axis=1)
    feats = _pool(xp, emb_table)
    return _mlp(feats, W1, b1, W2, b2, W3, b3)
